# baseline (speedup 1.0000x reference)
import functools

import numpy as np
import jax
import jax.numpy as jnp
from jax import lax
from jax.experimental import pallas as pl
from jax.experimental.pallas import tpu as pltpu
from jax.experimental.pallas import tpu_sc as plsc

NEG = -1e30
TOPK = 256
BLK_T = 256

_NC, _NS = 2, 16
_NW = _NC * _NS

_B_NEG = int(np.asarray(NEG, np.float32).view(np.int32))
_KEY_NEG = int(np.int32(_B_NEG) ^ np.int32(0x7FFFFFFF))
_MININT = -2147483648


def _monotone_i32(x_f32):
    b = lax.bitcast_convert_type(x_f32, jnp.int32)
    mask = lax.shift_right_arithmetic(b, 31) & jnp.int32(0x7FFFFFFF)
    return b ^ mask


def _proj_kernel(x_ref, w_ref, b_ref, o_ref):
    o_ref[...] = (
        jnp.dot(x_ref[...], w_ref[...], preferred_element_type=jnp.float32)
        + b_ref[...]
    )


def _proj_call(x2d, wcat, bcat):
    T, D = x2d.shape
    N = wcat.shape[1]
    return pl.pallas_call(
        _proj_kernel,
        grid=(T // BLK_T,),
        in_specs=[
            pl.BlockSpec((BLK_T, D), lambda i: (i, 0)),
            pl.BlockSpec((D, N), lambda i: (0, 0)),
            pl.BlockSpec((1, N), lambda i: (0, 0)),
        ],
        out_specs=pl.BlockSpec((BLK_T, N), lambda i: (i, 0)),
        out_shape=jax.ShapeDtypeStruct((T, N), jnp.float32),
    )(x2d, wcat, bcat)


def _scores_kernel(qi_ref, ki_ref, w_ref, i_ref, *, hi, di):
    i = pl.program_id(0)
    T = ki_ref.shape[0]
    acc = jnp.zeros((BLK_T, T), jnp.float32)
    for h in range(hi):
        qh = qi_ref[:, h * di:(h + 1) * di]
        kh = ki_ref[:, h * di:(h + 1) * di]
        sh = lax.dot_general(
            qh, kh, (((1,), (1,)), ((), ())),
            preferred_element_type=jnp.float32)
        acc = acc + w_ref[0, h] * jnp.maximum(sh, 0.0)
    t_glob = i * BLK_T + lax.broadcasted_iota(jnp.int32, (BLK_T, T), 0)
    s_idx = lax.broadcasted_iota(jnp.int32, (BLK_T, T), 1)
    i_ref[...] = jnp.where(s_idx <= t_glob, acc, NEG)


def _scores_call(qi, ki, w2d, hi, di):
    T = qi.shape[0]
    kern = functools.partial(_scores_kernel, hi=hi, di=di)
    return pl.pallas_call(
        kern,
        grid=(T // BLK_T,),
        in_specs=[
            pl.BlockSpec((BLK_T, hi * di), lambda i: (i, 0)),
            pl.BlockSpec((T, hi * di), lambda i: (0, 0)),
            pl.BlockSpec(memory_space=pltpu.SMEM),
        ],
        out_specs=pl.BlockSpec((BLK_T, T), lambda i: (i, 0)),
        out_shape=jax.ShapeDtypeStruct((T, T), jnp.float32),
    )(qi, ki, w2d)


def _sc_find_digit(hist, kr):
    def loop_a(jj, carry):
        found, jstar, runb, run = carry
        j = 15 - jj
        v = hist[pl.ds(j * 16, 16)]
        sj = jnp.sum(v)
        upd = (found == 0) & ((run + sj) >= kr)
        jstar = jnp.where(upd, j, jstar)
        runb = jnp.where(upd, run, runb)
        found = found | jnp.where(upd, jnp.int32(1), jnp.int32(0))
        return found, jstar, runb, run + sj

    z = jnp.int32(0)
    _, jstar, runb, _ = lax.fori_loop(0, 16, loop_a, (z, z, z, z))
    v = hist[pl.ds(jstar * 16, 16)]
    rv = lax.rev(v, (0,))
    c = plsc.cumsum(rv) + runb
    ge = c >= kr
    cs = plsc.cumsum(jnp.where(ge, jnp.int32(1), jnp.int32(0)))
    first = ge & (cs == 1)
    io = lax.iota(jnp.int32, 16)
    lane = jnp.sum(jnp.where(first, io, 0))
    cum_at = jnp.sum(jnp.where(first, c, 0))
    cnt_bucket = jnp.sum(jnp.where(first, rv, 0))
    digit = jstar * 16 + 15 - lane
    return digit, cum_at - cnt_bucket, cnt_bucket


def _sc_select_kernel(i_hbm, out_hbm, rowa_v, rowb_v, bufa, bufb, hist, outst,
                      sema, semb, *, row0, rpw):
    wid = lax.axis_index("s") * _NC + lax.axis_index("c")
    minint = jnp.int32(_MININT)

    def zero_hist():
        @plsc.parallel_loop(0, 16, unroll=4)
        def _(j):
            hist[pl.ds(j * 16, 16)] = jnp.zeros((16,), jnp.int32)

    def hist_sweep(src, nv, shift):
        @plsc.parallel_loop(0, nv, unroll=4)
        def _(i):
            ukey = src[pl.ds(i * 16, 16)]
            d = lax.shift_right_logical(ukey, shift) & jnp.int32(0xFF)
            cnt, last = plsc.scan_count(d)
            plsc.addupdate_scatter(hist, [d], cnt, mask=last)

    def compact(src, dst, nv, digit, shift):
        def body(i, off):
            ukey = src[pl.ds(i * 16, 16)]
            d = lax.shift_right_logical(ukey, shift) & jnp.int32(0xFF)
            msk = d == digit
            plsc.store_compressed(dst.at[pl.ds(off, 16)], ukey, mask=msk)
            return off + jnp.sum(jnp.where(msk, jnp.int32(1), jnp.int32(0)))
        off = lax.fori_loop(0, nv, body, jnp.int32(0))
        dst[pl.ds(off, 16)] = jnp.zeros((16,), jnp.int32)
        return off

    def process_row(r, row_v):
        t = row0 + wid + 32 * r
        nv0 = (t + 16) // 16

        zero_hist()

        @plsc.parallel_loop(0, nv0, unroll=4)
        def _(i):
            x = row_v[pl.ds(i * 16, 16)]
            b = lax.bitcast_convert_type(x, jnp.int32)
            mk = lax.shift_right_arithmetic(b, 31) & jnp.int32(0x7FFFFFFF)
            ukey = (b ^ mk) ^ minint
            bufa[pl.ds(i * 16, 16)] = ukey
            d = lax.shift_right_logical(ukey, 24) & jnp.int32(0xFF)
            cnt, last = plsc.scan_count(d)
            plsc.addupdate_scatter(hist, [d], cnt, mask=last)

        kr = jnp.int32(TOPK)
        d0, ca, _ = _sc_find_digit(hist, kr)
        prefix = lax.shift_left(d0, 24)
        kr = kr - ca
        nc = compact(bufa, bufb, nv0, d0, 24)
        nv = (nc + 15) // 16

        zero_hist()
        hist_sweep(bufb, nv, 16)
        d1, ca, _ = _sc_find_digit(hist, kr)
        prefix = prefix | lax.shift_left(d1, 16)
        kr = kr - ca
        nc = compact(bufb, bufa, nv, d1, 16)
        nv = (nc + 15) // 16

        zero_hist()
        hist_sweep(bufa, nv, 8)
        d2, ca, _ = _sc_find_digit(hist, kr)
        prefix = prefix | lax.shift_left(d2, 8)
        kr = kr - ca
        nc = compact(bufa, bufb, nv, d2, 8)
        nv = (nc + 15) // 16

        zero_hist()
        hist_sweep(bufb, nv, 0)
        d3, ca, _ = _sc_find_digit(hist, kr)
        prefix = prefix | d3
        kr = kr - ca

        thr = prefix ^ minint
        io = lax.iota(jnp.int32, 16)
        vec = jnp.where(io == 0, thr, jnp.where(io == 1, kr, jnp.int32(0)))
        outst[pl.ds(r * 16, 16)] = vec

    npairs = rpw // 2
    pltpu.async_copy(i_hbm.at[row0 + wid], rowa_v, sema)

    def pair_body(p, _):
        r0 = 2 * p
        t0 = row0 + wid + 32 * r0
        pltpu.make_async_copy(i_hbm.at[0], rowa_v, sema).wait()
        pltpu.async_copy(i_hbm.at[t0 + 32], rowb_v, semb)
        process_row(r0, rowa_v)
        pltpu.make_async_copy(i_hbm.at[0], rowb_v, semb).wait()

        @pl.when(p < npairs - 1)
        def _():
            pltpu.async_copy(i_hbm.at[t0 + 64], rowa_v, sema)

        process_row(r0 + 1, rowb_v)
        return 0

    lax.fori_loop(0, npairs, pair_body, 0)
    pltpu.sync_copy(outst, out_hbm.at[pl.ds(wid * (rpw * 16), rpw * 16)])


def _select_call(iscores, row0, nrows):
    T = iscores.shape[0]
    rpw = nrows // _NW
    mesh = plsc.VectorSubcoreMesh(
        core_axis_name="c", subcore_axis_name="s",
        num_cores=_NC, num_subcores=_NS)
    fn = pl.kernel(
        functools.partial(_sc_select_kernel, row0=row0, rpw=rpw),
        out_type=jax.ShapeDtypeStruct((nrows * 16,), jnp.int32),
        mesh=mesh,
        compiler_params=pltpu.CompilerParams(needs_layout_passes=False),
        scratch_types=[
            pltpu.VMEM((T,), jnp.float32),
            pltpu.VMEM((T,), jnp.float32),
            pltpu.VMEM((T + 16,), jnp.int32),
            pltpu.VMEM((T + 16,), jnp.int32),
            pltpu.VMEM((256,), jnp.int32),
            pltpu.VMEM((rpw * 16,), jnp.int32),
            pltpu.SemaphoreType.DMA,
            pltpu.SemaphoreType.DMA,
        ],
    )
    out = fn(iscores).reshape(_NW, rpw, 16)
    return out.transpose(1, 0, 2).reshape(nrows, 16)


def _attn_kernel(i_ref, tm_ref, q_ref, k_ref, v_ref, wo_ref, bo_ref,
                 o_ref, ctx_ref, *, nh, dh):
    T = k_ref.shape[0]
    scores = i_ref[...]
    s = _monotone_i32(scores)
    thr = tm_ref[:, 0:1]
    m = tm_ref[:, 1:2]
    eq = (s == thr).astype(jnp.int32)
    r = eq
    sh = 1
    while sh < T:
        r = r + jnp.concatenate(
            [jnp.zeros((BLK_T, sh), jnp.int32), r[:, :-sh]], axis=1)
        sh *= 2
    sel = (s > thr) | ((eq == 1) & (r <= m))
    sel = sel & (scores > NEG / 2)

    i_masked = jnp.where(sel, scores, NEG)
    mi = jnp.max(i_masked, axis=1, keepdims=True)
    e = jnp.exp(i_masked - mi)
    zi = jnp.sum(e, axis=1, keepdims=True)

    scale = 1.0 / (dh ** 0.5)
    for h in range(nh):
        qh = q_ref[:, h * dh:(h + 1) * dh] * scale
        kh = k_ref[:, h * dh:(h + 1) * dh]
        logits = lax.dot_general(
            qh, kh, (((1,), (1,)), ((), ())),
            preferred_element_type=jnp.float32)
        lm = jnp.where(sel, logits, NEG)
        ml = jnp.max(lm, axis=1, keepdims=True)
        p = jnp.exp(lm - ml)
        zl = jnp.sum(p, axis=1, keepdims=True)
        ctx_ref[:, h * dh:(h + 1) * dh] = jnp.dot(
            p * e, v_ref[:, h * dh:(h + 1) * dh],
            preferred_element_type=jnp.float32) / (zl * zi)
    o_ref[...] = (
        jnp.dot(ctx_ref[...], wo_ref[...], preferred_element_type=jnp.float32)
        + bo_ref[...]
    )


def _attn_call(iscores, thrm, q2d, k2d, v2d, wout, bout2d, nh, dh,
               row0, nrows):
    T, D = q2d.shape
    kern = functools.partial(_attn_kernel, nh=nh, dh=dh)
    off = row0 // BLK_T
    return pl.pallas_call(
        kern,
        grid=(nrows // BLK_T,),
        in_specs=[
            pl.BlockSpec((BLK_T, T), lambda i: (i + off, 0)),
            pl.BlockSpec((BLK_T, 16), lambda i: (i, 0)),
            pl.BlockSpec((BLK_T, D), lambda i: (i + off, 0)),
            pl.BlockSpec((T, D), lambda i: (0, 0)),
            pl.BlockSpec((T, D), lambda i: (0, 0)),
            pl.BlockSpec((D, D), lambda i: (0, 0)),
            pl.BlockSpec((1, D), lambda i: (0, 0)),
        ],
        out_specs=pl.BlockSpec((BLK_T, D), lambda i: (i, 0)),
        out_shape=jax.ShapeDtypeStruct((nrows, D), jnp.float32),
        scratch_shapes=[pltpu.VMEM((BLK_T, D), jnp.float32)],
    )(iscores, thrm, q2d, k2d, v2d, wout, bout2d)


def kernel(x, w_ih, Wq_idx, bq_idx, Wk_idx, bk_idx, Wqkv, bqkv, Wout, bout):
    B, T, D = x.shape
    HIDI = Wq_idx.shape[1]
    hi = w_ih.shape[0]
    di = HIDI // hi
    dh = 64
    nh = D // dh

    x2d = x.reshape(T, D)
    wcat = jnp.concatenate([Wqkv, Wq_idx, Wk_idx], axis=1)
    bcat = jnp.concatenate([bqkv, bq_idx, bk_idx], axis=0).reshape(1, -1)

    proj = _proj_call(x2d, wcat, bcat)
    q2d = proj[:, 0:D]
    k2d = proj[:, D:2 * D]
    v2d = proj[:, 2 * D:3 * D]
    qi = proj[:, 3 * D:3 * D + HIDI]
    ki = proj[:, 3 * D + HIDI:3 * D + 2 * HIDI]

    iscores = _scores_call(qi, ki, w_ih.reshape(1, hi), hi, di)

    io = jnp.arange(16, dtype=jnp.int32)
    thrm0 = jnp.broadcast_to(
        jnp.where(io == 0, jnp.int32(_KEY_NEG), jnp.int32(0)), (TOPK, 16))
    half = T // 2
    sel_a = _select_call(iscores, TOPK, half - TOPK)
    sel_b = _select_call(iscores, half, half)
    bout2d = bout.reshape(1, D)
    y_a = _attn_call(iscores, jnp.concatenate([thrm0, sel_a], axis=0),
                     q2d, k2d, v2d, Wout, bout2d, nh, dh, 0, half)
    y_b = _attn_call(iscores, sel_b,
                     q2d, k2d, v2d, Wout, bout2d, nh, dh, half, half)
    return jnp.concatenate([y_a, y_b], axis=0).reshape(B, T, D)


# speedup vs baseline: 1.2811x; 1.2811x over previous
import functools

import numpy as np
import jax
import jax.numpy as jnp
from jax import lax
from jax.experimental import pallas as pl
from jax.experimental.pallas import tpu as pltpu
from jax.experimental.pallas import tpu_sc as plsc

NEG = -1e30
TOPK = 256
BLK_T = 256

_NC, _NS = 2, 16
_NW = _NC * _NS

_B_NEG = int(np.asarray(NEG, np.float32).view(np.int32))
_KEY_NEG = int(np.int32(_B_NEG) ^ np.int32(0x7FFFFFFF))
_MININT = -2147483648


def _monotone_i32(x_f32):
    b = lax.bitcast_convert_type(x_f32, jnp.int32)
    mask = lax.shift_right_arithmetic(b, 31) & jnp.int32(0x7FFFFFFF)
    return b ^ mask


def _proj_kernel(x_ref, w1_ref, w2_ref, w3_ref, b_ref, o_ref):
    x = x_ref[...]
    d1 = w1_ref.shape[1]
    d2 = w2_ref.shape[1]
    d3 = w3_ref.shape[1]
    o_ref[:, 0:d1] = (
        jnp.dot(x, w1_ref[...], preferred_element_type=jnp.float32)
        + b_ref[:, 0:d1])
    o_ref[:, d1:d1 + d2] = (
        jnp.dot(x, w2_ref[...], preferred_element_type=jnp.float32)
        + b_ref[:, d1:d1 + d2])
    o_ref[:, d1 + d2:d1 + d2 + d3] = (
        jnp.dot(x, w3_ref[...], preferred_element_type=jnp.float32)
        + b_ref[:, d1 + d2:d1 + d2 + d3])


def _proj_call(x2d, w1, w2, w3, bcat):
    T, D = x2d.shape
    N = w1.shape[1] + w2.shape[1] + w3.shape[1]
    return pl.pallas_call(
        _proj_kernel,
        grid=(T // BLK_T,),
        in_specs=[
            pl.BlockSpec((BLK_T, D), lambda i: (i, 0)),
            pl.BlockSpec((D, w1.shape[1]), lambda i: (0, 0)),
            pl.BlockSpec((D, w2.shape[1]), lambda i: (0, 0)),
            pl.BlockSpec((D, w3.shape[1]), lambda i: (0, 0)),
            pl.BlockSpec((1, N), lambda i: (0, 0)),
        ],
        out_specs=pl.BlockSpec((BLK_T, N), lambda i: (i, 0)),
        out_shape=jax.ShapeDtypeStruct((T, N), jnp.float32),
    )(x2d, w1, w2, w3, bcat)


def _scores_kernel(qi_ref, ki_ref, w_ref, i_ref, *, hi, di):
    i = pl.program_id(0)
    T = ki_ref.shape[0]
    acc = jnp.zeros((BLK_T, T), jnp.float32)
    for h in range(hi):
        qh = qi_ref[:, h * di:(h + 1) * di]
        kh = ki_ref[:, h * di:(h + 1) * di]
        sh = lax.dot_general(
            qh, kh, (((1,), (1,)), ((), ())),
            preferred_element_type=jnp.float32)
        acc = acc + w_ref[0, h] * jnp.maximum(sh, 0.0)
    t_glob = i * BLK_T + lax.broadcasted_iota(jnp.int32, (BLK_T, T), 0)
    s_idx = lax.broadcasted_iota(jnp.int32, (BLK_T, T), 1)
    i_ref[...] = jnp.where(s_idx <= t_glob, acc, NEG)


def _scores_call(proj, w2d, hi, di, nh):
    T = proj.shape[0]
    hd = hi * di
    qi_blk = 3 * nh * 64 // hd
    kern = functools.partial(_scores_kernel, hi=hi, di=di)
    return pl.pallas_call(
        kern,
        grid=(T // BLK_T,),
        in_specs=[
            pl.BlockSpec((BLK_T, hd), lambda i: (i, qi_blk)),
            pl.BlockSpec((T, hd), lambda i: (0, qi_blk + 1)),
            pl.BlockSpec(memory_space=pltpu.SMEM),
        ],
        out_specs=pl.BlockSpec((BLK_T, T), lambda i: (i, 0)),
        out_shape=jax.ShapeDtypeStruct((T, T), jnp.float32),
    )(proj, proj, w2d)


def _sc_find_digit(hist, kr):
    def loop_a(jj, carry):
        found, jstar, runb, run = carry
        j = 15 - jj
        v = hist[pl.ds(j * 16, 16)]
        sj = jnp.sum(v)
        upd = (found == 0) & ((run + sj) >= kr)
        jstar = jnp.where(upd, j, jstar)
        runb = jnp.where(upd, run, runb)
        found = found | jnp.where(upd, jnp.int32(1), jnp.int32(0))
        return found, jstar, runb, run + sj

    z = jnp.int32(0)
    _, jstar, runb, _ = lax.fori_loop(0, 16, loop_a, (z, z, z, z))
    v = hist[pl.ds(jstar * 16, 16)]
    rv = lax.rev(v, (0,))
    c = plsc.cumsum(rv) + runb
    ge = c >= kr
    cs = plsc.cumsum(jnp.where(ge, jnp.int32(1), jnp.int32(0)))
    first = ge & (cs == 1)
    io = lax.iota(jnp.int32, 16)
    lane = jnp.sum(jnp.where(first, io, 0))
    cum_at = jnp.sum(jnp.where(first, c, 0))
    cnt_bucket = jnp.sum(jnp.where(first, rv, 0))
    digit = jstar * 16 + 15 - lane
    return digit, cum_at - cnt_bucket, cnt_bucket


def _sc_select_kernel(i_hbm, out_hbm, rowa_v, rowb_v, bufa, bufb, hist, outst,
                      sema, semb, *, row0, rpw):
    wid = lax.axis_index("s") * _NC + lax.axis_index("c")
    minint = jnp.int32(_MININT)

    def zero_hist():
        @plsc.parallel_loop(0, 16, unroll=4)
        def _(j):
            hist[pl.ds(j * 16, 16)] = jnp.zeros((16,), jnp.int32)

    def hist_sweep(src, nv, shift):
        @plsc.parallel_loop(0, nv, unroll=4)
        def _(i):
            ukey = src[pl.ds(i * 16, 16)]
            d = lax.shift_right_logical(ukey, shift) & jnp.int32(0xFF)
            cnt, last = plsc.scan_count(d)
            plsc.addupdate_scatter(hist, [d], cnt, mask=last)

    def compact(src, dst, nv, digit, shift):
        def body(i, off):
            ukey = src[pl.ds(i * 16, 16)]
            d = lax.shift_right_logical(ukey, shift) & jnp.int32(0xFF)
            msk = d == digit
            plsc.store_compressed(dst.at[pl.ds(off, 16)], ukey, mask=msk)
            return off + jnp.sum(jnp.where(msk, jnp.int32(1), jnp.int32(0)))
        off = lax.fori_loop(0, nv, body, jnp.int32(0))
        dst[pl.ds(off, 16)] = jnp.zeros((16,), jnp.int32)
        return off

    def process_row(r, row_v):
        t = row0 + wid + 32 * r
        nv0 = (t + 16) // 16

        zero_hist()

        @plsc.parallel_loop(0, nv0, unroll=4)
        def _(i):
            x = row_v[pl.ds(i * 16, 16)]
            b = lax.bitcast_convert_type(x, jnp.int32)
            mk = lax.shift_right_arithmetic(b, 31) & jnp.int32(0x7FFFFFFF)
            ukey = (b ^ mk) ^ minint
            bufa[pl.ds(i * 16, 16)] = ukey
            d = lax.shift_right_logical(ukey, 24) & jnp.int32(0xFF)
            cnt, last = plsc.scan_count(d)
            plsc.addupdate_scatter(hist, [d], cnt, mask=last)

        kr = jnp.int32(TOPK)
        d0, ca, _ = _sc_find_digit(hist, kr)
        prefix = lax.shift_left(d0, 24)
        kr = kr - ca
        nc = compact(bufa, bufb, nv0, d0, 24)
        nv = (nc + 15) // 16

        zero_hist()
        hist_sweep(bufb, nv, 16)
        d1, ca, _ = _sc_find_digit(hist, kr)
        prefix = prefix | lax.shift_left(d1, 16)
        kr = kr - ca
        nc = compact(bufb, bufa, nv, d1, 16)
        nv = (nc + 15) // 16

        zero_hist()
        hist_sweep(bufa, nv, 8)
        d2, ca, _ = _sc_find_digit(hist, kr)
        prefix = prefix | lax.shift_left(d2, 8)
        kr = kr - ca
        nc = compact(bufa, bufb, nv, d2, 8)
        nv = (nc + 15) // 16

        zero_hist()
        hist_sweep(bufb, nv, 0)
        d3, ca, _ = _sc_find_digit(hist, kr)
        prefix = prefix | d3
        kr = kr - ca

        thr = prefix ^ minint
        io = lax.iota(jnp.int32, 16)
        vec = jnp.where(io == 0, thr, jnp.where(io == 1, kr, jnp.int32(0)))
        outst[pl.ds(r * 16, 16)] = vec

    npairs = rpw // 2
    pltpu.async_copy(i_hbm.at[row0 + wid], rowa_v, sema)

    def pair_body(p, _):
        r0 = 2 * p
        t0 = row0 + wid + 32 * r0
        pltpu.make_async_copy(i_hbm.at[0], rowa_v, sema).wait()
        pltpu.async_copy(i_hbm.at[t0 + 32], rowb_v, semb)
        process_row(r0, rowa_v)
        pltpu.make_async_copy(i_hbm.at[0], rowb_v, semb).wait()

        @pl.when(p < npairs - 1)
        def _():
            pltpu.async_copy(i_hbm.at[t0 + 64], rowa_v, sema)

        process_row(r0 + 1, rowb_v)
        return 0

    lax.fori_loop(0, npairs, pair_body, 0)
    pltpu.sync_copy(outst, out_hbm.at[pl.ds(wid * (rpw * 16), rpw * 16)])


def _select_call(iscores, row0, nrows):
    T = iscores.shape[0]
    rpw = nrows // _NW
    mesh = plsc.VectorSubcoreMesh(
        core_axis_name="c", subcore_axis_name="s",
        num_cores=_NC, num_subcores=_NS)
    fn = pl.kernel(
        functools.partial(_sc_select_kernel, row0=row0, rpw=rpw),
        out_type=jax.ShapeDtypeStruct((nrows * 16,), jnp.int32),
        mesh=mesh,
        compiler_params=pltpu.CompilerParams(needs_layout_passes=False),
        scratch_types=[
            pltpu.VMEM((T,), jnp.float32),
            pltpu.VMEM((T,), jnp.float32),
            pltpu.VMEM((T + 16,), jnp.int32),
            pltpu.VMEM((T + 16,), jnp.int32),
            pltpu.VMEM((256,), jnp.int32),
            pltpu.VMEM((rpw * 16,), jnp.int32),
            pltpu.SemaphoreType.DMA,
            pltpu.SemaphoreType.DMA,
        ],
    )
    out = fn(iscores).reshape(_NW, rpw, 16)
    return out.transpose(1, 0, 2).reshape(nrows, 16)


def _attn_kernel(i_ref, tm_ref, q_ref, k_ref, v_ref, wo_ref, bo_ref,
                 o_ref, ctx_ref, *, nh, dh):
    T = k_ref.shape[0]
    scores = i_ref[...]
    s = _monotone_i32(scores)
    thr = tm_ref[:, 0:1]
    m = tm_ref[:, 1:2]
    eq = s == thr
    eqf = eq.astype(jnp.float32)
    io_r = lax.broadcasted_iota(jnp.int32, (BLK_T, BLK_T), 0)
    io_c = lax.broadcasted_iota(jnp.int32, (BLK_T, BLK_T), 1)
    tri = (io_r <= io_c).astype(jnp.float32)
    parts = []
    carry = jnp.zeros((BLK_T, 1), jnp.float32)
    for c in range(T // BLK_T):
        eqc = eqf[:, c * BLK_T:(c + 1) * BLK_T]
        rc = jnp.dot(eqc, tri, preferred_element_type=jnp.float32) + carry
        parts.append(rc)
        carry = rc[:, BLK_T - 1:BLK_T]
    rank = jnp.concatenate(parts, axis=1)
    sel = (s > thr) | (eq & (rank <= m.astype(jnp.float32)))
    sel = sel & (scores > NEG / 2)

    i_masked = jnp.where(sel, scores, NEG)
    mi = jnp.max(i_masked, axis=1, keepdims=True)
    e = jnp.exp(i_masked - mi)
    zi = jnp.sum(e, axis=1, keepdims=True)

    scale = 1.0 / (dh ** 0.5)
    for h in range(nh):
        qh = q_ref[:, h * dh:(h + 1) * dh] * scale
        kh = k_ref[:, h * dh:(h + 1) * dh]
        logits = lax.dot_general(
            qh, kh, (((1,), (1,)), ((), ())),
            preferred_element_type=jnp.float32)
        lm = jnp.where(sel, logits, NEG)
        ml = jnp.max(lm, axis=1, keepdims=True)
        p = jnp.exp(lm - ml)
        zl = jnp.sum(p, axis=1, keepdims=True)
        ctx_ref[:, h * dh:(h + 1) * dh] = jnp.dot(
            p * e, v_ref[:, h * dh:(h + 1) * dh],
            preferred_element_type=jnp.float32) / (zl * zi)
    o_ref[...] = (
        jnp.dot(ctx_ref[...], wo_ref[...], preferred_element_type=jnp.float32)
        + bo_ref[...]
    )


def _attn_call(iscores, thrm, proj, wout, bout2d, nh, dh):
    T = iscores.shape[0]
    D = nh * dh
    kern = functools.partial(_attn_kernel, nh=nh, dh=dh)
    return pl.pallas_call(
        kern,
        grid=(T // BLK_T,),
        in_specs=[
            pl.BlockSpec((BLK_T, T), lambda i: (i, 0)),
            pl.BlockSpec((BLK_T, 16), lambda i: (i, 0)),
            pl.BlockSpec((BLK_T, D), lambda i: (i, 0)),
            pl.BlockSpec((T, D), lambda i: (0, 1)),
            pl.BlockSpec((T, D), lambda i: (0, 2)),
            pl.BlockSpec((D, D), lambda i: (0, 0)),
            pl.BlockSpec((1, D), lambda i: (0, 0)),
        ],
        out_specs=pl.BlockSpec((BLK_T, D), lambda i: (i, 0)),
        out_shape=jax.ShapeDtypeStruct((T, D), jnp.float32),
        scratch_shapes=[pltpu.VMEM((BLK_T, D), jnp.float32)],
    )(iscores, thrm, proj, proj, proj, wout, bout2d)


def kernel(x, w_ih, Wq_idx, bq_idx, Wk_idx, bk_idx, Wqkv, bqkv, Wout, bout):
    B, T, D = x.shape
    HIDI = Wq_idx.shape[1]
    hi = w_ih.shape[0]
    di = HIDI // hi
    dh = 64
    nh = D // dh

    x2d = x.reshape(T, D)
    bcat = jnp.concatenate([bqkv, bq_idx, bk_idx], axis=0).reshape(1, -1)

    proj = _proj_call(x2d, Wqkv, Wq_idx, Wk_idx, bcat)
    iscores = _scores_call(proj, w_ih.reshape(1, hi), hi, di, nh)

    io = jnp.arange(16, dtype=jnp.int32)
    thrm0 = jnp.broadcast_to(
        jnp.where(io == 0, jnp.int32(_KEY_NEG), jnp.int32(0)), (TOPK, 16))
    sel = _select_call(iscores, TOPK, T - TOPK)
    thrm = jnp.concatenate([thrm0, sel], axis=0)
    y = _attn_call(iscores, thrm, proj, Wout, bout.reshape(1, D), nh, dh)
    return y.reshape(B, T, D)


# speedup vs baseline: 1.2863x; 1.0041x over previous
import functools

import numpy as np
import jax
import jax.numpy as jnp
from jax import lax
from jax.experimental import pallas as pl
from jax.experimental.pallas import tpu as pltpu
from jax.experimental.pallas import tpu_sc as plsc

NEG = -1e30
TOPK = 256
BLK_T = 256

_NC, _NS = 2, 16
_NW = _NC * _NS

_B_NEG = int(np.asarray(NEG, np.float32).view(np.int32))
_KEY_NEG = int(np.int32(_B_NEG) ^ np.int32(0x7FFFFFFF))
_MININT = -2147483648


def _monotone_i32(x_f32):
    b = lax.bitcast_convert_type(x_f32, jnp.int32)
    mask = lax.shift_right_arithmetic(b, 31) & jnp.int32(0x7FFFFFFF)
    return b ^ mask


def _proj_kernel(x_ref, w1_ref, w2_ref, w3_ref, b_ref, o_ref):
    x = x_ref[...]
    d1 = w1_ref.shape[1]
    d2 = w2_ref.shape[1]
    d3 = w3_ref.shape[1]
    o_ref[:, 0:d1] = (
        jnp.dot(x, w1_ref[...], preferred_element_type=jnp.float32)
        + b_ref[:, 0:d1])
    o_ref[:, d1:d1 + d2] = (
        jnp.dot(x, w2_ref[...], preferred_element_type=jnp.float32)
        + b_ref[:, d1:d1 + d2])
    o_ref[:, d1 + d2:d1 + d2 + d3] = (
        jnp.dot(x, w3_ref[...], preferred_element_type=jnp.float32)
        + b_ref[:, d1 + d2:d1 + d2 + d3])


def _proj_call(x2d, w1, w2, w3, bcat):
    T, D = x2d.shape
    N = w1.shape[1] + w2.shape[1] + w3.shape[1]
    return pl.pallas_call(
        _proj_kernel,
        grid=(T // BLK_T,),
        in_specs=[
            pl.BlockSpec((BLK_T, D), lambda i: (i, 0)),
            pl.BlockSpec((D, w1.shape[1]), lambda i: (0, 0)),
            pl.BlockSpec((D, w2.shape[1]), lambda i: (0, 0)),
            pl.BlockSpec((D, w3.shape[1]), lambda i: (0, 0)),
            pl.BlockSpec((1, N), lambda i: (0, 0)),
        ],
        out_specs=pl.BlockSpec((BLK_T, N), lambda i: (i, 0)),
        out_shape=jax.ShapeDtypeStruct((T, N), jnp.float32),
    )(x2d, w1, w2, w3, bcat)


def _scores_kernel(qi_ref, ki_ref, w_ref, i_ref, *, hi, di):
    i = pl.program_id(0)
    T = ki_ref.shape[0]
    acc = jnp.zeros((BLK_T, T), jnp.float32)
    for h in range(hi):
        qh = qi_ref[:, h * di:(h + 1) * di]
        kh = ki_ref[:, h * di:(h + 1) * di]
        sh = lax.dot_general(
            qh, kh, (((1,), (1,)), ((), ())),
            preferred_element_type=jnp.float32)
        acc = acc + w_ref[0, h] * jnp.maximum(sh, 0.0)
    t_glob = i * BLK_T + lax.broadcasted_iota(jnp.int32, (BLK_T, T), 0)
    s_idx = lax.broadcasted_iota(jnp.int32, (BLK_T, T), 1)
    i_ref[...] = jnp.where(s_idx <= t_glob, acc, NEG)


def _scores_call(proj, w2d, hi, di, nh):
    T = proj.shape[0]
    hd = hi * di
    qi_blk = 3 * nh * 64 // hd
    kern = functools.partial(_scores_kernel, hi=hi, di=di)
    return pl.pallas_call(
        kern,
        grid=(T // BLK_T,),
        in_specs=[
            pl.BlockSpec((BLK_T, hd), lambda i: (i, qi_blk)),
            pl.BlockSpec((T, hd), lambda i: (0, qi_blk + 1)),
            pl.BlockSpec(memory_space=pltpu.SMEM),
        ],
        out_specs=pl.BlockSpec((BLK_T, T), lambda i: (i, 0)),
        out_shape=jax.ShapeDtypeStruct((T, T), jnp.float32),
    )(proj, proj, w2d)


def _sc_find_digit(hist, kr):
    def loop_a(jj, carry):
        found, jstar, runb, run = carry
        j = 15 - jj
        v = hist[pl.ds(j * 16, 16)]
        sj = jnp.sum(v)
        upd = (found == 0) & ((run + sj) >= kr)
        jstar = jnp.where(upd, j, jstar)
        runb = jnp.where(upd, run, runb)
        found = found | jnp.where(upd, jnp.int32(1), jnp.int32(0))
        return found, jstar, runb, run + sj

    z = jnp.int32(0)
    _, jstar, runb, _ = lax.fori_loop(0, 16, loop_a, (z, z, z, z))
    v = hist[pl.ds(jstar * 16, 16)]
    rv = lax.rev(v, (0,))
    c = plsc.cumsum(rv) + runb
    ge = c >= kr
    cs = plsc.cumsum(jnp.where(ge, jnp.int32(1), jnp.int32(0)))
    first = ge & (cs == 1)
    io = lax.iota(jnp.int32, 16)
    lane = jnp.sum(jnp.where(first, io, 0))
    cum_at = jnp.sum(jnp.where(first, c, 0))
    cnt_bucket = jnp.sum(jnp.where(first, rv, 0))
    digit = jstar * 16 + 15 - lane
    return digit, cum_at - cnt_bucket, cnt_bucket


def _sc_select_kernel(i_hbm, out_hbm, rowa_v, rowb_v, bufa, bufb, hist, outst,
                      sema, semb, *, row0, rpw):
    wid = lax.axis_index("s") * _NC + lax.axis_index("c")
    minint = jnp.int32(_MININT)

    def zero_hist():
        @plsc.parallel_loop(0, 16, unroll=4)
        def _(j):
            hist[pl.ds(j * 16, 16)] = jnp.zeros((16,), jnp.int32)

    def hist_sweep(src, nv, shift):
        @plsc.parallel_loop(0, nv, unroll=4)
        def _(i):
            ukey = src[pl.ds(i * 16, 16)]
            d = lax.shift_right_logical(ukey, shift) & jnp.int32(0xFF)
            cnt, last = plsc.scan_count(d)
            plsc.addupdate_scatter(hist, [d], cnt, mask=last)

    def compact(src, dst, nv, digit, shift):
        @plsc.parallel_loop(0, nv, unroll=4, carry=jnp.int32(0))
        def body(i, off):
            ukey = src[pl.ds(i * 16, 16)]
            d = lax.shift_right_logical(ukey, shift) & jnp.int32(0xFF)
            msk = d == digit
            plsc.store_compressed(dst.at[pl.ds(off, 16)], ukey, mask=msk)
            return off + jnp.sum(jnp.where(msk, jnp.int32(1), jnp.int32(0)))
        off = body
        dst[pl.ds(off, 16)] = jnp.zeros((16,), jnp.int32)
        return off

    def process_row(r, row_v):
        t = row0 + wid + 32 * r
        nv0 = (t + 16) // 16

        zero_hist()

        @plsc.parallel_loop(0, nv0, unroll=8)
        def _(i):
            x = row_v[pl.ds(i * 16, 16)]
            b = lax.bitcast_convert_type(x, jnp.int32)
            mk = lax.shift_right_arithmetic(b, 31) & jnp.int32(0x7FFFFFFF)
            ukey = (b ^ mk) ^ minint
            bufa[pl.ds(i * 16, 16)] = ukey
            d = lax.shift_right_logical(ukey, 24) & jnp.int32(0xFF)
            cnt, last = plsc.scan_count(d)
            plsc.addupdate_scatter(hist, [d], cnt, mask=last)

        kr = jnp.int32(TOPK)
        d0, ca, _ = _sc_find_digit(hist, kr)
        prefix = lax.shift_left(d0, 24)
        kr = kr - ca
        nc = compact(bufa, bufb, nv0, d0, 24)
        nv = (nc + 15) // 16

        zero_hist()
        hist_sweep(bufb, nv, 16)
        d1, ca, _ = _sc_find_digit(hist, kr)
        prefix = prefix | lax.shift_left(d1, 16)
        kr = kr - ca
        nc = compact(bufb, bufa, nv, d1, 16)
        nv = (nc + 15) // 16

        zero_hist()
        hist_sweep(bufa, nv, 8)
        d2, ca, _ = _sc_find_digit(hist, kr)
        prefix = prefix | lax.shift_left(d2, 8)
        kr = kr - ca
        nc = compact(bufa, bufb, nv, d2, 8)
        nv = (nc + 15) // 16

        zero_hist()
        hist_sweep(bufb, nv, 0)
        d3, ca, _ = _sc_find_digit(hist, kr)
        prefix = prefix | d3
        kr = kr - ca

        thr = prefix ^ minint
        io = lax.iota(jnp.int32, 16)
        vec = jnp.where(io == 0, thr, jnp.where(io == 1, kr, jnp.int32(0)))
        outst[pl.ds(r * 16, 16)] = vec

    npairs = rpw // 2
    pltpu.async_copy(i_hbm.at[row0 + wid], rowa_v, sema)

    def pair_body(p, _):
        r0 = 2 * p
        t0 = row0 + wid + 32 * r0
        pltpu.make_async_copy(i_hbm.at[0], rowa_v, sema).wait()
        pltpu.async_copy(i_hbm.at[t0 + 32], rowb_v, semb)
        process_row(r0, rowa_v)
        pltpu.make_async_copy(i_hbm.at[0], rowb_v, semb).wait()

        @pl.when(p < npairs - 1)
        def _():
            pltpu.async_copy(i_hbm.at[t0 + 64], rowa_v, sema)

        process_row(r0 + 1, rowb_v)
        return 0

    lax.fori_loop(0, npairs, pair_body, 0)
    pltpu.sync_copy(outst, out_hbm.at[pl.ds(wid * (rpw * 16), rpw * 16)])


def _select_call(iscores, row0, nrows):
    T = iscores.shape[0]
    rpw = nrows // _NW
    mesh = plsc.VectorSubcoreMesh(
        core_axis_name="c", subcore_axis_name="s",
        num_cores=_NC, num_subcores=_NS)
    fn = pl.kernel(
        functools.partial(_sc_select_kernel, row0=row0, rpw=rpw),
        out_type=jax.ShapeDtypeStruct((nrows * 16,), jnp.int32),
        mesh=mesh,
        compiler_params=pltpu.CompilerParams(needs_layout_passes=False),
        scratch_types=[
            pltpu.VMEM((T,), jnp.float32),
            pltpu.VMEM((T,), jnp.float32),
            pltpu.VMEM((T + 16,), jnp.int32),
            pltpu.VMEM((T + 16,), jnp.int32),
            pltpu.VMEM((256,), jnp.int32),
            pltpu.VMEM((rpw * 16,), jnp.int32),
            pltpu.SemaphoreType.DMA,
            pltpu.SemaphoreType.DMA,
        ],
    )
    out = fn(iscores).reshape(_NW, rpw, 16)
    return out.transpose(1, 0, 2).reshape(nrows, 16)


def _attn_kernel(i_ref, tm_ref, q_ref, k_ref, v_ref, wo_ref, bo_ref,
                 o_ref, ctx_ref, *, nh, dh):
    T = k_ref.shape[0]
    scores = i_ref[...]
    s = _monotone_i32(scores)
    thr = tm_ref[:, 0:1]
    m = tm_ref[:, 1:2]
    eq = s == thr
    eqf = eq.astype(jnp.float32)
    io_r = lax.broadcasted_iota(jnp.int32, (BLK_T, BLK_T), 0)
    io_c = lax.broadcasted_iota(jnp.int32, (BLK_T, BLK_T), 1)
    tri = (io_r <= io_c).astype(jnp.float32)
    parts = []
    carry = jnp.zeros((BLK_T, 1), jnp.float32)
    for c in range(T // BLK_T):
        eqc = eqf[:, c * BLK_T:(c + 1) * BLK_T]
        rc = jnp.dot(eqc, tri, preferred_element_type=jnp.float32) + carry
        parts.append(rc)
        carry = rc[:, BLK_T - 1:BLK_T]
    rank = jnp.concatenate(parts, axis=1)
    sel = (s > thr) | (eq & (rank <= m.astype(jnp.float32)))
    sel = sel & (scores > NEG / 2)

    i_masked = jnp.where(sel, scores, NEG)
    mi = jnp.max(i_masked, axis=1, keepdims=True)
    e = jnp.exp(i_masked - mi)
    zi = jnp.sum(e, axis=1, keepdims=True)

    scale = 1.0 / (dh ** 0.5)
    for h in range(nh):
        qh = q_ref[:, h * dh:(h + 1) * dh] * scale
        kh = k_ref[:, h * dh:(h + 1) * dh]
        logits = lax.dot_general(
            qh, kh, (((1,), (1,)), ((), ())),
            preferred_element_type=jnp.float32)
        lm = jnp.where(sel, logits, NEG)
        ml = jnp.max(lm, axis=1, keepdims=True)
        p = jnp.exp(lm - ml)
        zl = jnp.sum(p, axis=1, keepdims=True)
        ctx_ref[:, h * dh:(h + 1) * dh] = jnp.dot(
            p * e, v_ref[:, h * dh:(h + 1) * dh],
            preferred_element_type=jnp.float32) / (zl * zi)
    o_ref[...] = (
        jnp.dot(ctx_ref[...], wo_ref[...], preferred_element_type=jnp.float32)
        + bo_ref[...]
    )


def _attn_call(iscores, thrm, proj, wout, bout2d, nh, dh):
    T = iscores.shape[0]
    D = nh * dh
    kern = functools.partial(_attn_kernel, nh=nh, dh=dh)
    return pl.pallas_call(
        kern,
        grid=(T // BLK_T,),
        in_specs=[
            pl.BlockSpec((BLK_T, T), lambda i: (i, 0)),
            pl.BlockSpec((BLK_T, 16), lambda i: (i, 0)),
            pl.BlockSpec((BLK_T, D), lambda i: (i, 0)),
            pl.BlockSpec((T, D), lambda i: (0, 1)),
            pl.BlockSpec((T, D), lambda i: (0, 2)),
            pl.BlockSpec((D, D), lambda i: (0, 0)),
            pl.BlockSpec((1, D), lambda i: (0, 0)),
        ],
        out_specs=pl.BlockSpec((BLK_T, D), lambda i: (i, 0)),
        out_shape=jax.ShapeDtypeStruct((T, D), jnp.float32),
        scratch_shapes=[pltpu.VMEM((BLK_T, D), jnp.float32)],
    )(iscores, thrm, proj, proj, proj, wout, bout2d)


def kernel(x, w_ih, Wq_idx, bq_idx, Wk_idx, bk_idx, Wqkv, bqkv, Wout, bout):
    B, T, D = x.shape
    HIDI = Wq_idx.shape[1]
    hi = w_ih.shape[0]
    di = HIDI // hi
    dh = 64
    nh = D // dh

    x2d = x.reshape(T, D)
    bcat = jnp.concatenate([bqkv, bq_idx, bk_idx], axis=0).reshape(1, -1)

    proj = _proj_call(x2d, Wqkv, Wq_idx, Wk_idx, bcat)
    iscores = _scores_call(proj, w_ih.reshape(1, hi), hi, di, nh)

    io = jnp.arange(16, dtype=jnp.int32)
    thrm0 = jnp.broadcast_to(
        jnp.where(io == 0, jnp.int32(_KEY_NEG), jnp.int32(0)), (TOPK, 16))
    sel = _select_call(iscores, TOPK, T - TOPK)
    thrm = jnp.concatenate([thrm0, sel], axis=0)
    y = _attn_call(iscores, thrm, proj, Wout, bout.reshape(1, D), nh, dh)
    return y.reshape(B, T, D)
